# baseline (speedup 1.0000x reference)
import functools

import jax
import jax.numpy as jnp
from jax import lax
from jax.experimental import pallas as pl
from jax.experimental.pallas import tpu as pltpu
from jax.experimental.pallas import tpu_sc as plsc

N_TOK = 8192
IN_F = 4096
OUT_F = 4096
FAN_IN = 32
L = 16


@functools.lru_cache(maxsize=1)
def _make_densify():
    info = plsc.get_sparse_core_info()
    nw = info.num_cores * info.num_subcores
    o_per_w = OUT_F // nw
    grp = 8
    ngrp = o_per_w // grp
    mesh = plsc.VectorSubcoreMesh(core_axis_name="c", subcore_axis_name="s")

    @functools.partial(
        pl.kernel,
        mesh=mesh,
        out_type=jax.ShapeDtypeStruct((OUT_F * IN_F,), jnp.float32),
        compiler_params=pltpu.CompilerParams(needs_layout_passes=False),
        scratch_types=[
            pltpu.VMEM((o_per_w, FAN_IN), jnp.int32),
            pltpu.VMEM((o_per_w, FAN_IN), jnp.float32),
            pltpu.VMEM((grp * IN_F,), jnp.float32),
        ],
    )
    def densify(mask_hbm, w_hbm, out_hbm, mask_v, w_v, buf):
        wid = lax.axis_index("s") * info.num_cores + lax.axis_index("c")
        o_base = wid * o_per_w
        pltpu.sync_copy(mask_hbm.at[pl.ds(o_base, o_per_w)], mask_v)
        pltpu.sync_copy(w_hbm.at[pl.ds(o_base, o_per_w)], w_v)

        zeros = jnp.zeros((L,), jnp.float32)

        def zbody(i, c):
            buf[pl.ds(i * L, L)] = zeros
            return c

        lax.fori_loop(0, (grp * IN_F) // L, zbody, 0)

        for g in range(ngrp):
            for r in range(grp):
                ol = g * grp + r
                for h in range(FAN_IN // L):
                    idx = mask_v[ol, pl.ds(h * L, L)] + r * IN_F
                    val = w_v[ol, pl.ds(h * L, L)]
                    plsc.addupdate_scatter(buf, [idx], val)
            pltpu.sync_copy(
                buf, out_hbm.at[pl.ds((o_base + g * grp) * IN_F, grp * IN_F)]
            )
            for r in range(grp):
                ol = g * grp + r
                for h in range(FAN_IN // L):
                    idx = mask_v[ol, pl.ds(h * L, L)] + r * IN_F
                    plsc.store_scatter(buf, [idx], zeros)

    return densify


BN = 2048
BO = 512
BK = 4096


def _mm_body(x_ref, w_ref, b_ref, o_ref):
    xb = x_ref[...]
    wb = w_ref[...]
    o_ref[...] = jnp.broadcast_to(b_ref[...], (BN, BO)) + lax.dot_general(
        xb, wb, (((1,), (1,)), ((), ())), preferred_element_type=jnp.float32
    )


def _matmul(x, wdense, bias2d):
    grid = (N_TOK // BN, OUT_F // BO)
    return pl.pallas_call(
        _mm_body,
        grid=grid,
        in_specs=[
            pl.BlockSpec((BN, BK), lambda n, o: (n, 0)),
            pl.BlockSpec((BO, BK), lambda n, o: (o, 0)),
            pl.BlockSpec((1, BO), lambda n, o: (0, o)),
        ],
        out_specs=pl.BlockSpec((BN, BO), lambda n, o: (n, o)),
        out_shape=jax.ShapeDtypeStruct((N_TOK, OUT_F), jnp.float32),
        compiler_params=pltpu.CompilerParams(
            dimension_semantics=("parallel", "parallel"),
            vmem_limit_bytes=128 * 1024 * 1024,
        ),
    )(x, wdense, bias2d)


def kernel(input, condensed_weight, input_mask, bias):
    densify = _make_densify()
    wdense = densify(input_mask, condensed_weight).reshape(OUT_F, IN_F)
    wdense = wdense.astype(jnp.bfloat16)
    return _matmul(input.astype(jnp.bfloat16), wdense, bias.reshape(1, OUT_F))


# speedup vs baseline: 1.0151x; 1.0151x over previous
import functools

import jax
import jax.numpy as jnp
from jax import lax
from jax.experimental import pallas as pl
from jax.experimental.pallas import tpu as pltpu
from jax.experimental.pallas import tpu_sc as plsc

N_TOK = 8192
IN_F = 4096
OUT_F = 4096
FAN_IN = 32
L = 16


@functools.lru_cache(maxsize=1)
def _make_densify():
    info = plsc.get_sparse_core_info()
    nw = info.num_cores * info.num_subcores
    o_per_w = OUT_F // nw
    grp = 8
    ngrp = o_per_w // grp
    mesh = plsc.VectorSubcoreMesh(core_axis_name="c", subcore_axis_name="s")

    @functools.partial(
        pl.kernel,
        mesh=mesh,
        out_type=jax.ShapeDtypeStruct((OUT_F * IN_F,), jnp.float32),
        compiler_params=pltpu.CompilerParams(needs_layout_passes=False),
        scratch_types=[
            pltpu.VMEM((o_per_w, FAN_IN), jnp.int32),
            pltpu.VMEM((o_per_w, FAN_IN), jnp.float32),
            pltpu.VMEM((grp * IN_F,), jnp.float32),
        ],
    )
    def densify(mask_hbm, w_hbm, out_hbm, mask_v, w_v, buf):
        wid = lax.axis_index("s") * info.num_cores + lax.axis_index("c")
        o_base = wid * o_per_w
        pltpu.sync_copy(mask_hbm.at[pl.ds(o_base, o_per_w)], mask_v)
        pltpu.sync_copy(w_hbm.at[pl.ds(o_base, o_per_w)], w_v)

        zeros = jnp.zeros((L,), jnp.float32)

        def zbody(i, c):
            buf[pl.ds(i * L, L)] = zeros
            return c

        lax.fori_loop(0, (grp * IN_F) // L, zbody, 0)

        for g in range(ngrp):
            for r in range(grp):
                ol = g * grp + r
                for h in range(FAN_IN // L):
                    idx = mask_v[ol, pl.ds(h * L, L)] + r * IN_F
                    val = w_v[ol, pl.ds(h * L, L)]
                    plsc.addupdate_scatter(buf, [idx], val)
            pltpu.sync_copy(
                buf, out_hbm.at[pl.ds((o_base + g * grp) * IN_F, grp * IN_F)]
            )
            for r in range(grp):
                ol = g * grp + r
                for h in range(FAN_IN // L):
                    idx = mask_v[ol, pl.ds(h * L, L)] + r * IN_F
                    plsc.store_scatter(buf, [idx], zeros)

    return densify


BN = 1024
BO = 512
BK = 4096


def _mm_body(x_ref, w_ref, b_ref, o_ref, xbf_ref):
    @pl.when(pl.program_id(1) == 0)
    def _():
        xbf_ref[...] = x_ref[...].astype(jnp.bfloat16)

    o_ref[...] = jnp.broadcast_to(b_ref[...], (BN, BO)) + lax.dot_general(
        xbf_ref[...], w_ref[...], (((1,), (1,)), ((), ())),
        preferred_element_type=jnp.float32,
    )


def _matmul(x, wdense, bias2d):
    grid = (N_TOK // BN, OUT_F // BO)
    return pl.pallas_call(
        _mm_body,
        grid=grid,
        in_specs=[
            pl.BlockSpec((BN, BK), lambda n, o: (n, 0)),
            pl.BlockSpec((BO, BK), lambda n, o: (o, 0)),
            pl.BlockSpec((1, BO), lambda n, o: (0, o)),
        ],
        out_specs=pl.BlockSpec((BN, BO), lambda n, o: (n, o)),
        out_shape=jax.ShapeDtypeStruct((N_TOK, OUT_F), jnp.float32),
        scratch_shapes=[pltpu.VMEM((BN, BK), jnp.bfloat16)],
        compiler_params=pltpu.CompilerParams(
            dimension_semantics=("parallel", "parallel"),
            vmem_limit_bytes=128 * 1024 * 1024,
        ),
    )(x, wdense, bias2d)


def kernel(input, condensed_weight, input_mask, bias):
    densify = _make_densify()
    wdense = densify(input_mask, condensed_weight).reshape(OUT_F, IN_F)
    wdense = wdense.astype(jnp.bfloat16)
    return _matmul(input, wdense, bias.reshape(1, OUT_F))


# speedup vs baseline: 1.1024x; 1.0860x over previous
import functools

import jax
import jax.numpy as jnp
from jax import lax
from jax.experimental import pallas as pl
from jax.experimental.pallas import tpu as pltpu
from jax.experimental.pallas import tpu_sc as plsc

N_TOK = 8192
IN_F = 4096
OUT_F = 4096
FAN_IN = 32
L = 16


@functools.lru_cache(maxsize=1)
def _make_densify():
    info = plsc.get_sparse_core_info()
    nw = info.num_cores * info.num_subcores
    o_per_w = OUT_F // nw
    grp = 8
    ngrp = o_per_w // grp
    mesh = plsc.VectorSubcoreMesh(core_axis_name="c", subcore_axis_name="s")

    @functools.partial(
        pl.kernel,
        mesh=mesh,
        out_type=jax.ShapeDtypeStruct((OUT_F * IN_F,), jnp.float32),
        compiler_params=pltpu.CompilerParams(needs_layout_passes=False),
        scratch_types=[
            pltpu.VMEM((o_per_w, FAN_IN), jnp.int32),
            pltpu.VMEM((o_per_w, FAN_IN), jnp.float32),
            pltpu.VMEM((grp * IN_F,), jnp.float32),
        ],
    )
    def densify(mask_hbm, w_hbm, out_hbm, mask_v, w_v, buf):
        wid = lax.axis_index("s") * info.num_cores + lax.axis_index("c")
        o_base = wid * o_per_w
        pltpu.sync_copy(mask_hbm.at[pl.ds(o_base, o_per_w)], mask_v)
        pltpu.sync_copy(w_hbm.at[pl.ds(o_base, o_per_w)], w_v)

        zeros = jnp.zeros((L,), jnp.float32)

        def zbody(i, c):
            buf[pl.ds(i * L, L)] = zeros
            return c

        lax.fori_loop(0, (grp * IN_F) // L, zbody, 0)

        for g in range(ngrp):
            for r in range(grp):
                ol = g * grp + r
                for h in range(FAN_IN // L):
                    idx = mask_v[ol, pl.ds(h * L, L)] + r * IN_F
                    val = w_v[ol, pl.ds(h * L, L)]
                    plsc.addupdate_scatter(buf, [idx], val)
            pltpu.sync_copy(
                buf, out_hbm.at[pl.ds((o_base + g * grp) * IN_F, grp * IN_F)]
            )
            for r in range(grp):
                ol = g * grp + r
                for h in range(FAN_IN // L):
                    idx = mask_v[ol, pl.ds(h * L, L)] + r * IN_F
                    plsc.store_scatter(buf, [idx], zeros)

    return densify


BN = 256


def _mm_body(x_ref, w_any, b_ref, o_ref, w_vmem, sem):
    @pl.when(pl.program_id(0) == 0)
    def _():
        cp = pltpu.make_async_copy(w_any, w_vmem, sem)
        cp.start()
        cp.wait()

    xb = x_ref[...].astype(jnp.bfloat16)
    o_ref[...] = jnp.broadcast_to(b_ref[...], (BN, OUT_F)) + lax.dot_general(
        xb, w_vmem[...], (((1,), (1,)), ((), ())),
        preferred_element_type=jnp.float32,
    )


def _matmul(x, wdense, bias2d):
    return pl.pallas_call(
        _mm_body,
        grid=(N_TOK // BN,),
        in_specs=[
            pl.BlockSpec((BN, IN_F), lambda n: (n, 0)),
            pl.BlockSpec(memory_space=pl.ANY),
            pl.BlockSpec((1, OUT_F), lambda n: (0, 0)),
        ],
        out_specs=pl.BlockSpec((BN, OUT_F), lambda n: (n, 0)),
        out_shape=jax.ShapeDtypeStruct((N_TOK, OUT_F), jnp.float32),
        scratch_shapes=[
            pltpu.VMEM((OUT_F, IN_F), jnp.bfloat16),
            pltpu.SemaphoreType.DMA,
        ],
        compiler_params=pltpu.CompilerParams(
            dimension_semantics=("arbitrary",),
            vmem_limit_bytes=128 * 1024 * 1024,
        ),
    )(x, wdense, bias2d)


def kernel(input, condensed_weight, input_mask, bias):
    densify = _make_densify()
    wdense = densify(input_mask, condensed_weight).reshape(OUT_F, IN_F)
    wdense = wdense.astype(jnp.bfloat16)
    return _matmul(input, wdense, bias.reshape(1, OUT_F))


# speedup vs baseline: 1.1116x; 1.0083x over previous
import functools

import jax
import jax.numpy as jnp
from jax import lax
from jax.experimental import pallas as pl
from jax.experimental.pallas import tpu as pltpu
from jax.experimental.pallas import tpu_sc as plsc

N_TOK = 8192
IN_F = 4096
OUT_F = 4096
FAN_IN = 32
L = 16


@functools.lru_cache(maxsize=1)
def _make_densify():
    info = plsc.get_sparse_core_info()
    nw = info.num_cores * info.num_subcores
    o_per_w = OUT_F // nw
    grp = 8
    ngrp = o_per_w // grp
    mesh = plsc.VectorSubcoreMesh(core_axis_name="c", subcore_axis_name="s")

    @functools.partial(
        pl.kernel,
        mesh=mesh,
        out_type=jax.ShapeDtypeStruct((OUT_F * IN_F,), jnp.float32),
        compiler_params=pltpu.CompilerParams(needs_layout_passes=False),
        scratch_types=[
            pltpu.VMEM((o_per_w, FAN_IN), jnp.int32),
            pltpu.VMEM((o_per_w, FAN_IN), jnp.float32),
            pltpu.VMEM((grp * IN_F,), jnp.float32),
        ],
    )
    def densify(mask_hbm, w_hbm, out_hbm, mask_v, w_v, buf):
        wid = lax.axis_index("s") * info.num_cores + lax.axis_index("c")
        o_base = wid * o_per_w
        pltpu.sync_copy(mask_hbm.at[pl.ds(o_base, o_per_w)], mask_v)
        pltpu.sync_copy(w_hbm.at[pl.ds(o_base, o_per_w)], w_v)

        zeros = jnp.zeros((L,), jnp.float32)

        def zbody(i, c):
            buf[pl.ds(i * L, L)] = zeros
            return c

        lax.fori_loop(0, (grp * IN_F) // L, zbody, 0)

        for g in range(ngrp):
            for r in range(grp):
                ol = g * grp + r
                for h in range(FAN_IN // L):
                    idx = mask_v[ol, pl.ds(h * L, L)] + r * IN_F
                    val = w_v[ol, pl.ds(h * L, L)]
                    plsc.addupdate_scatter(buf, [idx], val)
            pltpu.sync_copy(
                buf, out_hbm.at[pl.ds((o_base + g * grp) * IN_F, grp * IN_F)]
            )
            for r in range(grp):
                ol = g * grp + r
                for h in range(FAN_IN // L):
                    idx = mask_v[ol, pl.ds(h * L, L)] + r * IN_F
                    plsc.store_scatter(buf, [idx], zeros)

    return densify


BN = 256
WCH = 16
WCS = OUT_F // WCH


def _mm_body(x_ref, w_any, b_ref, o_ref, w_vmem, wstg, sems):
    @pl.when(pl.program_id(0) == 0)
    def _():
        def cp(c):
            return pltpu.make_async_copy(
                w_any.at[pl.ds(c * WCS, WCS)], wstg.at[c % 2], sems.at[c % 2]
            )

        cp(0).start()
        for c in range(WCH):
            if c + 1 < WCH:
                cp(c + 1).start()
            cp(c).wait()
            w_vmem[pl.ds(c * WCS, WCS), :] = wstg[c % 2].astype(jnp.bfloat16)

    xb = x_ref[...].astype(jnp.bfloat16)
    o_ref[...] = jnp.broadcast_to(b_ref[...], (BN, OUT_F)) + lax.dot_general(
        xb, w_vmem[...], (((1,), (1,)), ((), ())),
        preferred_element_type=jnp.float32,
    )


def _matmul(x, wdense, bias2d):
    return pl.pallas_call(
        _mm_body,
        grid=(N_TOK // BN,),
        in_specs=[
            pl.BlockSpec((BN, IN_F), lambda n: (n, 0)),
            pl.BlockSpec(memory_space=pl.ANY),
            pl.BlockSpec((1, OUT_F), lambda n: (0, 0)),
        ],
        out_specs=pl.BlockSpec((BN, OUT_F), lambda n: (n, 0)),
        out_shape=jax.ShapeDtypeStruct((N_TOK, OUT_F), jnp.float32),
        scratch_shapes=[
            pltpu.VMEM((OUT_F, IN_F), jnp.bfloat16),
            pltpu.VMEM((2, WCS, IN_F), jnp.float32),
            pltpu.SemaphoreType.DMA((2,)),
        ],
        compiler_params=pltpu.CompilerParams(
            dimension_semantics=("arbitrary",),
            vmem_limit_bytes=128 * 1024 * 1024,
        ),
    )(x, wdense, bias2d)


def kernel(input, condensed_weight, input_mask, bias):
    densify = _make_densify()
    wdense = densify(input_mask, condensed_weight).reshape(OUT_F, IN_F)
    return _matmul(input, wdense, bias.reshape(1, OUT_F))
